# baseline (speedup 1.0000x reference)
import functools

import jax
import jax.numpy as jnp
from jax import lax
from jax.experimental import pallas as pl

ROWS = 4 * 4096
D = 2048
BLOCK_ROWS = 512
_THRESH = 1258291


def _threefry_mask(rows_u32):
    ks0 = jnp.uint32(0)
    ks1 = jnp.uint32(42)
    ks2 = jnp.uint32(0x1BD11BDA ^ 42)
    ks = (ks0, ks1, ks2)
    rot_a = (13, 15, 26, 6)
    rot_b = (17, 29, 16, 24)

    x0 = jnp.zeros_like(rows_u32) + ks0
    x1 = rows_u32 + ks1
    for g in range(5):
        for r in (rot_a if g % 2 == 0 else rot_b):
            x0 = x0 + x1
            x1 = (x1 << r) | (x1 >> (32 - r))
            x1 = x1 ^ x0
        x0 = x0 + ks[(g + 1) % 3]
        x1 = x1 + ks[(g + 2) % 3] + jnp.uint32(g + 1)
    bits = x0 ^ x1
    return (bits >> 9) <= jnp.uint32(_THRESH)


def _body(x_ref, mv_ref, o_ref):
    i = pl.program_id(0)
    rows = jnp.uint32(i * BLOCK_ROWS) + lax.broadcasted_iota(
        jnp.uint32, (BLOCK_ROWS, 1), 0)
    m = _threefry_mask(rows)
    o_ref[...] = jnp.where(m, mv_ref[...], x_ref[...])


@jax.jit
def kernel(inputs, mask_value):
    x = inputs.reshape(ROWS, D)
    mv = mask_value.reshape(1, D)
    out = pl.pallas_call(
        _body,
        grid=(ROWS // BLOCK_ROWS,),
        in_specs=[
            pl.BlockSpec((BLOCK_ROWS, D), lambda i: (i, 0)),
            pl.BlockSpec((1, D), lambda i: (0, 0)),
        ],
        out_specs=pl.BlockSpec((BLOCK_ROWS, D), lambda i: (i, 0)),
        out_shape=jax.ShapeDtypeStruct((ROWS, D), jnp.float32),
    )(x, mv)
    return out.reshape(inputs.shape)
